# baseline (speedup 1.0000x reference)
import functools

import jax
import jax.numpy as jnp
from jax import lax
from jax.experimental import pallas as pl
from jax.experimental.pallas import tpu as pltpu
from jax.experimental.pallas import tpu_sc as plsc

B = 16384
D = 64
L = 16
NC = 2
NS = 16
NW = NC * NS
RPW = B // NW
CH = 128
NCHUNK = RPW // CH


def _sc_body(e1_hbm, rel_hbm, e2_hbm, er_hbm, ei_hbm, rr_hbm, ri_hbm,
             out_hbm,
             e1_v, rel_v, e2_v,
             a_r, a_i, r_r, r_i, b_r, b_i,
             s_v, out_v, sem):
    wid = lax.axis_index("s") * NC + lax.axis_index("c")
    row0 = wid * RPW

    def chunk_body(c, carry):
        base = row0 + c * CH
        pltpu.sync_copy(e1_hbm.at[pl.ds(base, CH)], e1_v)
        pltpu.sync_copy(rel_hbm.at[pl.ds(base, CH)], rel_v)
        pltpu.sync_copy(e2_hbm.at[pl.ds(base, CH)], e2_v)
        cps = [
            pltpu.async_copy(er_hbm.at[e1_v], a_r, sem),
            pltpu.async_copy(ei_hbm.at[e1_v], a_i, sem),
            pltpu.async_copy(rr_hbm.at[rel_v], r_r, sem),
            pltpu.async_copy(ri_hbm.at[rel_v], r_i, sem),
            pltpu.async_copy(er_hbm.at[e2_v], b_r, sem),
            pltpu.async_copy(ei_hbm.at[e2_v], b_i, sem),
        ]
        for cp in cps:
            cp.wait()

        def group_body(g, carry2):
            def row_body(r, carry3):
                row = g * L + r
                acc = jnp.zeros((L,), jnp.float32)
                for k in range(D // L):
                    sl = pl.ds(k * L, L)
                    ar = a_r[row, sl]
                    ai = a_i[row, sl]
                    rr = r_r[row, sl]
                    ri = r_i[row, sl]
                    br = b_r[row, sl]
                    bi = b_i[row, sl]
                    acc = acc + br * (ar * rr - ai * ri) + bi * (ar * ri + ai * rr)
                s_v[pl.ds(pl.multiple_of(r * L, L), L)] = acc
                return carry3

            lax.fori_loop(0, L, row_body, 0)
            lane = lax.iota(jnp.int32, L)
            tot = jnp.zeros((L,), jnp.float32)
            for j in range(L):
                tot = tot + plsc.load_gather(s_v, [lane * L + j])
            res = 1.0 / (1.0 + jnp.exp(-tot))
            off = pl.multiple_of(c * CH + g * L, L)
            out_v[pl.ds(off, L)] = res
            return carry2

        lax.fori_loop(0, CH // L, group_body, 0)
        return carry

    lax.fori_loop(0, NCHUNK, chunk_body, 0)
    pltpu.sync_copy(out_v, out_hbm.at[pl.ds(row0, RPW)])


@jax.jit
def _scores(e1_idx, rel_idx, e2_idx, ent_real, ent_img, rel_real, rel_img):
    mesh = plsc.VectorSubcoreMesh(core_axis_name="c", subcore_axis_name="s")
    fn = pl.kernel(
        _sc_body,
        mesh=mesh,
        compiler_params=pltpu.CompilerParams(
            needs_layout_passes=False, use_tc_tiling_on_sc=False
        ),
        out_type=jax.ShapeDtypeStruct((B,), jnp.float32),
        scratch_types=[
            pltpu.VMEM((CH,), jnp.int32),
            pltpu.VMEM((CH,), jnp.int32),
            pltpu.VMEM((CH,), jnp.int32),
            pltpu.VMEM((CH, D), jnp.float32),
            pltpu.VMEM((CH, D), jnp.float32),
            pltpu.VMEM((CH, D), jnp.float32),
            pltpu.VMEM((CH, D), jnp.float32),
            pltpu.VMEM((CH, D), jnp.float32),
            pltpu.VMEM((CH, D), jnp.float32),
            pltpu.VMEM((L * L,), jnp.float32),
            pltpu.VMEM((RPW,), jnp.float32),
            pltpu.SemaphoreType.DMA,
        ],
    )
    return fn(e1_idx, rel_idx, e2_idx, ent_real, ent_img, rel_real, rel_img)


def kernel(e1_idx, rel_idx, e2_idx, ent_real, ent_img, rel_real, rel_img):
    e1 = e1_idx.astype(jnp.int32)
    rel = rel_idx.astype(jnp.int32)
    e2 = e2_idx.astype(jnp.int32)
    out = _scores(e1, rel, e2, ent_real, ent_img, rel_real, rel_img)
    return (out, jnp.float32(0.0))
